# baseline (speedup 1.0000x reference)
import jax
import jax.numpy as jnp
from jax.experimental import pallas as pl

_BB = 4096


def _lookup_block(t_ref, hi_ref, out_ref):
    tb = t_ref[0, 0, :].reshape(_BB, 1)
    vp = hi_ref.shape[0]
    col = jax.lax.broadcasted_iota(jnp.int32, (_BB, vp), 1)
    oh = (tb == col).astype(jnp.bfloat16)
    out_ref[:, :] = jnp.dot(oh, hi_ref[:], preferred_element_type=jnp.float32)


def kernel(t, embed):
    B = t.shape[0]
    V, D = embed.shape
    Vp = ((V + 127) // 128) * 128
    hi = jnp.pad(embed.astype(jnp.bfloat16), ((0, Vp - V), (0, 0)))
    nb = B // _BB
    t3 = t.astype(jnp.int32).reshape(nb, 1, _BB)
    return pl.pallas_call(
        _lookup_block,
        grid=(nb,),
        in_specs=[
            pl.BlockSpec((1, 1, _BB), lambda i: (i, 0, 0)),
            pl.BlockSpec((Vp, D), lambda i: (0, 0)),
        ],
        out_specs=pl.BlockSpec((_BB, D), lambda i: (i, 0)),
        out_shape=jax.ShapeDtypeStruct((B, D), jnp.float32),
    )(t3, hi)


# speedup vs baseline: 1.0523x; 1.0523x over previous
import jax
import jax.numpy as jnp
from jax.experimental import pallas as pl

_BB = 2048


def _lookup_block(t_ref, hi_ref, out_ref):
    tb = t_ref[0, 0, :].reshape(_BB, 1)
    vp = hi_ref.shape[0]
    col = jax.lax.broadcasted_iota(jnp.int32, (_BB, vp), 1)
    oh = (tb == col).astype(jnp.bfloat16)
    out_ref[:, :] = jnp.dot(oh, hi_ref[:], preferred_element_type=jnp.float32)


def kernel(t, embed):
    B = t.shape[0]
    V, D = embed.shape
    Vp = ((V + 127) // 128) * 128
    hi = jnp.pad(embed.astype(jnp.bfloat16), ((0, Vp - V), (0, 0)))
    nb = B // _BB
    t3 = t.astype(jnp.int32).reshape(nb, 1, _BB)
    return pl.pallas_call(
        _lookup_block,
        grid=(nb,),
        in_specs=[
            pl.BlockSpec((1, 1, _BB), lambda i: (i, 0, 0)),
            pl.BlockSpec((Vp, D), lambda i: (0, 0)),
        ],
        out_specs=pl.BlockSpec((_BB, D), lambda i: (i, 0)),
        out_shape=jax.ShapeDtypeStruct((B, D), jnp.float32),
    )(t3, hi)


# speedup vs baseline: 1.4376x; 1.3662x over previous
import jax
import jax.numpy as jnp
from jax.experimental import pallas as pl

_BB = 2048


def _lookup_block(t_ref, tbl_ref, out_ref):
    tb = t_ref[0, 0, :].reshape(_BB, 1)
    v = tbl_ref.shape[0]
    col = jax.lax.broadcasted_iota(jnp.int32, (_BB, v), 1)
    oh = (tb == col).astype(jnp.float32)
    out_ref[:, :] = jnp.dot(oh, tbl_ref[:], preferred_element_type=jnp.float32)


def kernel(t, embed):
    B = t.shape[0]
    V, D = embed.shape
    nb = B // _BB
    t3 = t.astype(jnp.int32).reshape(nb, 1, _BB)
    return pl.pallas_call(
        _lookup_block,
        grid=(nb,),
        in_specs=[
            pl.BlockSpec((1, 1, _BB), lambda i: (i, 0, 0)),
            pl.BlockSpec((V, D), lambda i: (0, 0)),
        ],
        out_specs=pl.BlockSpec((_BB, D), lambda i: (i, 0)),
        out_shape=jax.ShapeDtypeStruct((B, D), jnp.float32),
    )(t3, embed)
